# baseline (device time: 74557 ns/iter reference)
import jax
import jax.numpy as jnp
from jax import lax
from jax.experimental import pallas as pl
from jax.experimental.pallas import tpu as pltpu

N_DEV = 8
HQ_PER = 8
DH = 128
SQ = 256
SKV = 4096
BLK = 64
SCALE = 0.08838834764831843


def kernel(x, Wq, K_ext, V_ext, Wo):
    x2 = x.reshape(SQ, 1024)
    K = K_ext.reshape(SKV, 64, DH)
    V = V_ext.reshape(SKV, 64, DH)

    RS_ROWS = (128, 64, 32)
    RBUF_OFF = (0, 128, 192)

    def body(x_ref, wq_ref, k_hbm, v_hbm, wo_ref, out_ref,
             kbuf, vbuf, rbuf, copy_sems, rs_send, rs_recv, ag_send, ag_recv):
        my = lax.axis_index("i")
        bits = [(my >> k) & 1 for k in range(3)]
        partners = [my + (1 - 2 * bits[k]) * (1 << k) for k in range(3)]

        barrier = pltpu.get_barrier_semaphore()
        for p in partners:
            pl.semaphore_signal(barrier, 1, device_id=(p,),
                                device_id_type=pl.DeviceIdType.MESH)

        h0 = my * HQ_PER
        kcopies, vcopies = [], []
        for h in range(HQ_PER):
            ck = pltpu.make_async_copy(
                k_hbm.at[:, pl.ds(h0 + h, 1), :], kbuf.at[h],
                copy_sems.at[0, h])
            cv = pltpu.make_async_copy(
                v_hbm.at[:, pl.ds(h0 + h, 1), :], vbuf.at[h],
                copy_sems.at[1, h])
            ck.start()
            cv.start()
            kcopies.append(ck)
            vcopies.append(cv)

        q_all = jnp.dot(x_ref[...], wq_ref[...],
                        preferred_element_type=jnp.float32)

        qb = lax.broadcasted_iota(jnp.int32, (SQ, SKV), 0) // BLK
        kb = lax.broadcasted_iota(jnp.int32, (SQ, SKV), 1) // BLK
        mask = (qb == kb) | (kb == 0) | (lax.rem(qb + kb, 3) == 0)

        ctx_parts = []
        for h in range(HQ_PER):
            kcopies[h].wait()
            vcopies[h].wait()
            qh = q_all[:, h * DH:(h + 1) * DH]
            kh = kbuf[h, :, 0, :]
            s = lax.dot_general(
                qh, kh, (((1,), (1,)), ((), ())),
                preferred_element_type=jnp.float32) * SCALE
            s = jnp.where(mask, s, -1e9)
            m = jnp.max(s, axis=1, keepdims=True)
            e = jnp.exp(s - m)
            w = e / jnp.sum(e, axis=1, keepdims=True)
            ctx_parts.append(jnp.dot(w, vbuf[h, :, 0, :],
                                     preferred_element_type=jnp.float32))
        ctx = jnp.concatenate(ctx_parts, axis=1)
        partial = jnp.dot(ctx, wo_ref[...],
                          preferred_element_type=jnp.float32)

        out_ref[...] = partial

        pl.semaphore_wait(barrier, 3)

        COMM_OFF = False
        cur_start = my - my
        for k, half in (enumerate(RS_ROWS) if not COMM_OFF else []):
            send_start = cur_start + (1 - bits[k]) * half
            keep_start = cur_start + bits[k] * half
            rdma = pltpu.make_async_remote_copy(
                src_ref=out_ref.at[pl.ds(send_start, half), :],
                dst_ref=rbuf.at[pl.ds(RBUF_OFF[k], half), :],
                send_sem=rs_send.at[k],
                recv_sem=rs_recv.at[k],
                device_id=(partners[k],),
                device_id_type=pl.DeviceIdType.MESH,
            )
            rdma.start()
            rdma.wait()
            out_ref[pl.ds(keep_start, half), :] = (
                out_ref[pl.ds(keep_start, half), :]
                + rbuf[pl.ds(RBUF_OFF[k], half), :])
            cur_start = keep_start

        own_start, size = cur_start, RS_ROWS[-1]
        for k in ((2, 1, 0) if not COMM_OFF else []):
            rdma = pltpu.make_async_remote_copy(
                src_ref=out_ref.at[pl.ds(own_start, size), :],
                dst_ref=out_ref.at[pl.ds(own_start, size), :],
                send_sem=ag_send.at[k],
                recv_sem=ag_recv.at[k],
                device_id=(partners[k],),
                device_id_type=pl.DeviceIdType.MESH,
            )
            rdma.start()
            rdma.wait()
            own_start = own_start - bits[k] * size
            size *= 2

    out2 = pl.pallas_call(
        body,
        out_shape=jax.ShapeDtypeStruct((SQ, 1024), jnp.float32),
        in_specs=[
            pl.BlockSpec(memory_space=pltpu.VMEM),
            pl.BlockSpec(memory_space=pltpu.VMEM),
            pl.BlockSpec(memory_space=pl.ANY),
            pl.BlockSpec(memory_space=pl.ANY),
            pl.BlockSpec(memory_space=pltpu.VMEM),
        ],
        out_specs=pl.BlockSpec(memory_space=pltpu.VMEM),
        scratch_shapes=[
            pltpu.VMEM((HQ_PER, SKV, 1, DH), jnp.float32),
            pltpu.VMEM((HQ_PER, SKV, 1, DH), jnp.float32),
            pltpu.VMEM((224, 1024), jnp.float32),
            pltpu.SemaphoreType.DMA((2, HQ_PER)),
            pltpu.SemaphoreType.DMA((3,)),
            pltpu.SemaphoreType.DMA((3,)),
            pltpu.SemaphoreType.DMA((3,)),
            pltpu.SemaphoreType.DMA((3,)),
        ],
        compiler_params=pltpu.CompilerParams(
            collective_id=0,
            vmem_limit_bytes=100 * 1024 * 1024,
        ),
    )(x2, Wq, K, V, Wo)
    return out2.reshape(1, SQ, 1024)


# device time: 62314 ns/iter; 1.1965x vs baseline; 1.1965x over previous
import jax
import jax.numpy as jnp
from jax import lax
from jax.experimental import pallas as pl
from jax.experimental.pallas import tpu as pltpu

N_DEV = 8
HQ_PER = 8
DH = 128
SQ = 256
SKV = 4096
BLK = 64
ROWS = SQ // N_DEV
SCALE = 0.08838834764831843


def kernel(x, Wq, K_ext, V_ext, Wo):
    x2 = x.reshape(SQ, 1024)
    K = K_ext.reshape(SKV, 64, DH)
    V = V_ext.reshape(SKV, 64, DH)

    def body(x_ref, wq_ref, k_hbm, v_hbm, wo_ref, out_ref,
             kbuf, vbuf, rbuf, copy_sems, rs_send, rs_recv, ag_send, ag_recv):
        my = lax.axis_index("i")

        barrier = pltpu.get_barrier_semaphore()
        for d in range(1, N_DEV):
            pl.semaphore_signal(barrier, 1,
                                device_id=(lax.rem(my + d, N_DEV),),
                                device_id_type=pl.DeviceIdType.MESH)

        h0 = my * HQ_PER
        kcopies, vcopies = [], []
        for h in range(HQ_PER):
            ck = pltpu.make_async_copy(
                k_hbm.at[:, pl.ds(h0 + h, 1), :], kbuf.at[h],
                copy_sems.at[0, h])
            cv = pltpu.make_async_copy(
                v_hbm.at[:, pl.ds(h0 + h, 1), :], vbuf.at[h],
                copy_sems.at[1, h])
            ck.start()
            cv.start()
            kcopies.append(ck)
            vcopies.append(cv)

        q_all = jnp.dot(x_ref[...], wq_ref[...],
                        preferred_element_type=jnp.float32)

        qb = lax.broadcasted_iota(jnp.int32, (SQ, SKV), 0) // BLK
        kb = lax.broadcasted_iota(jnp.int32, (SQ, SKV), 1) // BLK
        mask = (qb == kb) | (kb == 0) | (lax.rem(qb + kb, 3) == 0)

        ctx_parts = []
        for h in range(HQ_PER):
            kcopies[h].wait()
            vcopies[h].wait()
            qh = q_all[:, h * DH:(h + 1) * DH]
            kh = kbuf[h, :, 0, :]
            s = lax.dot_general(
                qh, kh, (((1,), (1,)), ((), ())),
                preferred_element_type=jnp.float32) * SCALE
            s = jnp.where(mask, s, -1e9)
            m = jnp.max(s, axis=1, keepdims=True)
            e = jnp.exp(s - m)
            w = e / jnp.sum(e, axis=1, keepdims=True)
            ctx_parts.append(jnp.dot(w, vbuf[h, :, 0, :],
                                     preferred_element_type=jnp.float32))
        ctx = jnp.concatenate(ctx_parts, axis=1)
        out_ref[...] = jnp.dot(ctx, wo_ref[...],
                               preferred_element_type=jnp.float32)

        pl.semaphore_wait(barrier, N_DEV - 1)

        for d in range(1, N_DEV):
            tgt = lax.rem(my + d, N_DEV)
            pltpu.make_async_remote_copy(
                src_ref=out_ref.at[pl.ds(tgt * ROWS, ROWS), :],
                dst_ref=rbuf.at[pl.ds((d - 1) * ROWS, ROWS), :],
                send_sem=rs_send.at[d - 1],
                recv_sem=rs_recv.at[d - 1],
                device_id=(tgt,),
                device_id_type=pl.DeviceIdType.MESH,
            ).start()

        def dummy(tgt_ref, j, send_sems, recv_sems):
            return pltpu.make_async_remote_copy(
                src_ref=out_ref.at[pl.ds(j * ROWS, ROWS), :],
                dst_ref=tgt_ref.at[pl.ds(j * ROWS, ROWS), :],
                send_sem=send_sems.at[j],
                recv_sem=recv_sems.at[j],
                device_id=(my,),
                device_id_type=pl.DeviceIdType.MESH,
            )

        for j in range(N_DEV - 1):
            dummy(rbuf, j, rs_send, rs_recv).wait_recv()
        acc = out_ref[pl.ds(my * ROWS, ROWS), :]
        for j in range(N_DEV - 1):
            acc = acc + rbuf[j * ROWS:(j + 1) * ROWS, :]
        out_ref[pl.ds(my * ROWS, ROWS), :] = acc

        for d in range(1, N_DEV):
            pltpu.make_async_remote_copy(
                src_ref=out_ref.at[pl.ds(my * ROWS, ROWS), :],
                dst_ref=out_ref.at[pl.ds(my * ROWS, ROWS), :],
                send_sem=ag_send.at[d - 1],
                recv_sem=ag_recv.at[d - 1],
                device_id=(lax.rem(my + d, N_DEV),),
                device_id_type=pl.DeviceIdType.MESH,
            ).start()

        for j in range(N_DEV - 1):
            dummy(out_ref, j, rs_send, rs_recv).wait_send()
        for j in range(N_DEV - 1):
            dummy(out_ref, j, ag_send, ag_recv).wait_recv()
        for j in range(N_DEV - 1):
            dummy(out_ref, j, ag_send, ag_recv).wait_send()

    out2 = pl.pallas_call(
        body,
        out_shape=jax.ShapeDtypeStruct((SQ, 1024), jnp.float32),
        in_specs=[
            pl.BlockSpec(memory_space=pltpu.VMEM),
            pl.BlockSpec(memory_space=pltpu.VMEM),
            pl.BlockSpec(memory_space=pl.ANY),
            pl.BlockSpec(memory_space=pl.ANY),
            pl.BlockSpec(memory_space=pltpu.VMEM),
        ],
        out_specs=pl.BlockSpec(memory_space=pltpu.VMEM),
        scratch_shapes=[
            pltpu.VMEM((HQ_PER, SKV, 1, DH), jnp.float32),
            pltpu.VMEM((HQ_PER, SKV, 1, DH), jnp.float32),
            pltpu.VMEM(((N_DEV - 1) * ROWS, 1024), jnp.float32),
            pltpu.SemaphoreType.DMA((2, HQ_PER)),
            pltpu.SemaphoreType.DMA((N_DEV - 1,)),
            pltpu.SemaphoreType.DMA((N_DEV - 1,)),
            pltpu.SemaphoreType.DMA((N_DEV - 1,)),
            pltpu.SemaphoreType.DMA((N_DEV - 1,)),
        ],
        compiler_params=pltpu.CompilerParams(
            collective_id=0,
            vmem_limit_bytes=100 * 1024 * 1024,
        ),
    )(x2, Wq, K, V, Wo)
    return out2.reshape(1, SQ, 1024)


# device time: 43515 ns/iter; 1.7134x vs baseline; 1.4320x over previous
import jax
import jax.numpy as jnp
from jax import lax
from jax.experimental import pallas as pl
from jax.experimental.pallas import tpu as pltpu

N_DEV = 8
HQ_PER = 8
DH = 128
SQ = 256
SKV = 4096
BLK = 64
ROWS = SQ // N_DEV
SCALE = 0.08838834764831843


def kernel(x, Wq, K_ext, V_ext, Wo):
    x2 = x.reshape(SQ, 1024)
    K = K_ext.reshape(SKV, 64, DH)
    V = V_ext.reshape(SKV, 64, DH)

    def body(x_ref, wq_ref, k_hbm, v_hbm, wo_ref, out_ref,
             kbuf, vbuf, rbuf, copy_sems, rs_send, rs_recv, ag_send, ag_recv):
        my = lax.axis_index("i")

        barrier = pltpu.get_barrier_semaphore()
        for d in range(1, N_DEV):
            pl.semaphore_signal(barrier, 1,
                                device_id=(lax.rem(my + d, N_DEV),),
                                device_id_type=pl.DeviceIdType.MESH)

        h0 = my * HQ_PER
        kcopies, vcopies = [], []
        for h in range(HQ_PER):
            ck = pltpu.make_async_copy(
                k_hbm.at[:, h0 + h, :], kbuf.at[h],
                copy_sems.at[0, h])
            cv = pltpu.make_async_copy(
                v_hbm.at[:, h0 + h, :], vbuf.at[h],
                copy_sems.at[1, h])
            ck.start()
            cv.start()
            kcopies.append(ck)
            vcopies.append(cv)

        q_all = jnp.dot(x_ref[...], wq_ref[...],
                        preferred_element_type=jnp.float32) * SCALE

        qb = lax.broadcasted_iota(jnp.int32, (SQ, SKV), 0) // BLK
        kb = lax.broadcasted_iota(jnp.int32, (SQ, SKV), 1) // BLK
        mask = (qb == kb) | (kb == 0) | (lax.rem(qb + kb, 3) == 0)

        ctx_parts = []
        for h in range(HQ_PER):
            kcopies[h].wait()
            vcopies[h].wait()
            qh = q_all[:, h * DH:(h + 1) * DH]
            kh = kbuf[h]
            s = lax.dot_general(
                qh, kh, (((1,), (1,)), ((), ())),
                preferred_element_type=jnp.float32)
            e = jnp.exp(jnp.where(mask, s, -1e9))
            recip = 1.0 / jnp.sum(e, axis=1, keepdims=True)
            ctx_parts.append(jnp.dot(e, vbuf[h],
                                     preferred_element_type=jnp.float32)
                             * recip)
        ctx = jnp.concatenate(ctx_parts, axis=1)
        out_ref[...] = jnp.dot(ctx, wo_ref[...],
                               preferred_element_type=jnp.float32)

        pl.semaphore_wait(barrier, N_DEV - 1)

        for d in range(1, N_DEV):
            tgt = lax.rem(my + d, N_DEV)
            pltpu.make_async_remote_copy(
                src_ref=out_ref.at[pl.ds(tgt * ROWS, ROWS), :],
                dst_ref=rbuf.at[pl.ds((d - 1) * ROWS, ROWS), :],
                send_sem=rs_send.at[d - 1],
                recv_sem=rs_recv.at[d - 1],
                device_id=(tgt,),
                device_id_type=pl.DeviceIdType.MESH,
            ).start()

        def dummy(tgt_ref, j, send_sems, recv_sems):
            return pltpu.make_async_remote_copy(
                src_ref=out_ref.at[pl.ds(j * ROWS, ROWS), :],
                dst_ref=tgt_ref.at[pl.ds(j * ROWS, ROWS), :],
                send_sem=send_sems.at[j],
                recv_sem=recv_sems.at[j],
                device_id=(my,),
                device_id_type=pl.DeviceIdType.MESH,
            )

        for j in range(N_DEV - 1):
            dummy(rbuf, j, rs_send, rs_recv).wait_recv()
        acc = out_ref[pl.ds(my * ROWS, ROWS), :]
        for j in range(N_DEV - 1):
            acc = acc + rbuf[j * ROWS:(j + 1) * ROWS, :]
        out_ref[pl.ds(my * ROWS, ROWS), :] = acc

        for d in range(1, N_DEV):
            pltpu.make_async_remote_copy(
                src_ref=out_ref.at[pl.ds(my * ROWS, ROWS), :],
                dst_ref=out_ref.at[pl.ds(my * ROWS, ROWS), :],
                send_sem=ag_send.at[d - 1],
                recv_sem=ag_recv.at[d - 1],
                device_id=(lax.rem(my + d, N_DEV),),
                device_id_type=pl.DeviceIdType.MESH,
            ).start()

        for j in range(N_DEV - 1):
            dummy(out_ref, j, rs_send, rs_recv).wait_send()
        for j in range(N_DEV - 1):
            dummy(out_ref, j, ag_send, ag_recv).wait_recv()
        for j in range(N_DEV - 1):
            dummy(out_ref, j, ag_send, ag_recv).wait_send()

    out2 = pl.pallas_call(
        body,
        out_shape=jax.ShapeDtypeStruct((SQ, 1024), jnp.float32),
        in_specs=[
            pl.BlockSpec(memory_space=pltpu.VMEM),
            pl.BlockSpec(memory_space=pltpu.VMEM),
            pl.BlockSpec(memory_space=pl.ANY),
            pl.BlockSpec(memory_space=pl.ANY),
            pl.BlockSpec(memory_space=pltpu.VMEM),
        ],
        out_specs=pl.BlockSpec(memory_space=pltpu.VMEM),
        scratch_shapes=[
            pltpu.VMEM((HQ_PER, SKV, DH), jnp.float32),
            pltpu.VMEM((HQ_PER, SKV, DH), jnp.float32),
            pltpu.VMEM(((N_DEV - 1) * ROWS, 1024), jnp.float32),
            pltpu.SemaphoreType.DMA((2, HQ_PER)),
            pltpu.SemaphoreType.DMA((N_DEV - 1,)),
            pltpu.SemaphoreType.DMA((N_DEV - 1,)),
            pltpu.SemaphoreType.DMA((N_DEV - 1,)),
            pltpu.SemaphoreType.DMA((N_DEV - 1,)),
        ],
        compiler_params=pltpu.CompilerParams(
            collective_id=0,
            vmem_limit_bytes=100 * 1024 * 1024,
        ),
    )(x2, Wq, K, V, Wo)
    return out2.reshape(1, SQ, 1024)
